# baseline (speedup 1.0000x reference)
import functools

import jax
import jax.numpy as jnp
from jax import lax
from jax.experimental import pallas as pl
from jax.experimental.pallas import tpu as pltpu
from jax.experimental.pallas import tpu_sc as plsc

H = 1024
E = 8
K = 2
B = 2
S = 2048
N = B * S
CAP = int(1.25 * N * K / E)
ECAP = E * CAP
R = ECAP + 8
FFN = 1536
AUX = 0.01
P = N * K
WGT_W = 128

NC = 2
NS = 16
NW = NC * NS

TOK_BLK = 512
NB = N // TOK_BLK


def _routing_body(x_ref, g_ref, wr_ref, dst_ref, gat_ref, wpr_ref, cnt_ref,
                  aux_ref, xfb_ref, carry, mesum):
    i = pl.program_id(0)

    @pl.when(i == 0)
    def _init():
        carry[...] = jnp.zeros((1, E), jnp.float32)
        mesum[...] = jnp.zeros((1, E), jnp.float32)

    @pl.when(i < 2 * NB)
    def _main():
        kk = i // NB
        b = i % NB
        xb = x_ref[...]
        ms = jnp.mean(xb * xb, axis=1, keepdims=True)
        xn = xb * lax.rsqrt(ms + 1e-6) * g_ref[...]
        logits = jnp.dot(xn, wr_ref[...], preferred_element_type=jnp.float32)
        mx = jnp.max(logits, axis=1, keepdims=True)
        ex = jnp.exp(logits - mx)
        p = ex / jnp.sum(ex, axis=1, keepdims=True)
        iota8 = lax.broadcasted_iota(jnp.int32, (TOK_BLK, E), 1)
        m1 = jnp.max(p, axis=1, keepdims=True)
        i1 = jnp.min(jnp.where(p == m1, iota8, E), axis=1, keepdims=True)
        p2 = jnp.where(iota8 == i1, -jnp.inf, p)
        m2 = jnp.max(p2, axis=1, keepdims=True)
        i2 = jnp.min(jnp.where(p2 == m2, iota8, E), axis=1, keepdims=True)
        tot = m1 + m2
        wk = jnp.where(kk == 0, m1 / tot, m2 / tot)
        ek = jnp.where(kk == 0, i1, i2)
        oh = (iota8 == ek).astype(jnp.float32)
        r_io = lax.broadcasted_iota(jnp.int32, (TOK_BLK, TOK_BLK), 0)
        c_io = lax.broadcasted_iota(jnp.int32, (TOK_BLK, TOK_BLK), 1)
        tri = (r_io >= c_io).astype(jnp.float32)
        csum = jnp.dot(tri, oh, preferred_element_type=jnp.float32,
                       precision=lax.Precision.HIGHEST)
        base = carry[...]
        posm = csum - 1.0 + base
        pos = jnp.sum(oh * posm, axis=1, keepdims=True).astype(jnp.int32)
        assigned = pos < CAP
        slot = ek * CAP + pos
        off = kk * N + b * TOK_BLK
        dst_ref[pl.ds(off, TOK_BLK)] = jnp.where(assigned, slot, ECAP).reshape(TOK_BLK)
        gat_ref[pl.ds(off, TOK_BLK)] = jnp.where(assigned, slot, -1).reshape(TOK_BLK)
        wpr_ref[pl.ds(off, TOK_BLK), :] = jnp.broadcast_to(wk, (TOK_BLK, WGT_W))
        carry[...] = base + jnp.sum(oh, axis=0, keepdims=True)

        @pl.when(kk == 0)
        def _xb():
            xr = xb.astype(jnp.bfloat16).astype(jnp.float32)
            u = lax.bitcast_convert_type(xr, jnp.int32)
            lo = lax.shift_right_logical(u[:, :H // 2], 16)
            hi = u[:, H // 2:] & jnp.int32(-65536)
            xfb_ref[pl.ds(b * TOK_BLK, TOK_BLK), :] = lo | hi

        @pl.when(kk == 0)
        def _me():
            mesum[...] = mesum[...] + jnp.sum(p, axis=0, keepdims=True)

    @pl.when(i == 2 * NB)
    def _final():
        ctot = carry[...]
        cc = jnp.minimum(ctot, float(CAP))
        cnt_ref[...] = cc.astype(jnp.int32)
        mn = jnp.min(cc)
        e_io = lax.broadcasted_iota(jnp.int32, (1, E), 1)
        e0 = jnp.min(jnp.where(cc == mn, e_io, E))
        zslot = e0 * CAP + mn.astype(jnp.int32)
        g_all = gat_ref[...]
        gat_ref[...] = jnp.where(g_all < 0, zslot, g_all)
        me = mesum[...] / N
        ce = ctot / (N * K)
        aux_ref[...] = AUX * E * jnp.sum(me * ce, axis=1, keepdims=True)


def _routing(xf, g2d, Wr):
    grid = (2 * NB + 1,)
    return pl.pallas_call(
        _routing_body,
        grid=grid,
        in_specs=[
            pl.BlockSpec((TOK_BLK, H), lambda i: (i % NB, 0)),
            pl.BlockSpec((1, H), lambda i: (0, 0)),
            pl.BlockSpec((H, E), lambda i: (0, 0)),
        ],
        out_specs=[
            pl.BlockSpec((P,), lambda i: (0,)),
            pl.BlockSpec((P,), lambda i: (0,)),
            pl.BlockSpec((P, WGT_W), lambda i: (0, 0)),
            pl.BlockSpec((1, E), lambda i: (0, 0)),
            pl.BlockSpec((1, 1), lambda i: (0, 0)),
            pl.BlockSpec((N, H // 2), lambda i: (0, 0)),
        ],
        out_shape=[
            jax.ShapeDtypeStruct((P,), jnp.int32),
            jax.ShapeDtypeStruct((P,), jnp.int32),
            jax.ShapeDtypeStruct((P, WGT_W), jnp.float32),
            jax.ShapeDtypeStruct((1, E), jnp.int32),
            jax.ShapeDtypeStruct((1, 1), jnp.float32),
            jax.ShapeDtypeStruct((N, H // 2), jnp.int32),
        ],
        scratch_shapes=[
            pltpu.VMEM((1, E), jnp.float32),
            pltpu.VMEM((1, E), jnp.float32),
        ],
    )(xf, g2d, Wr)


PPW = P // NW
DCH = 64


def _dispatch(xfb, dst, wpr):
    mesh = plsc.VectorSubcoreMesh(core_axis_name="c", subcore_axis_name="s")
    NCH = PPW // DCH

    @functools.partial(
        pl.kernel,
        out_type=(jax.ShapeDtypeStruct((R, H // 2), jnp.int32),
                  jax.ShapeDtypeStruct((R, WGT_W), jnp.float32)),
        mesh=mesh,
        scratch_types=[
            pltpu.VMEM((DCH,), jnp.int32),
            pltpu.VMEM((DCH,), jnp.int32),
            pltpu.VMEM((DCH, H // 2), jnp.int32),
            pltpu.VMEM((DCH, H // 2), jnp.int32),
            pltpu.VMEM((DCH, WGT_W), jnp.float32),
            pltpu.VMEM((DCH, WGT_W), jnp.float32),
            pltpu.SemaphoreType.DMA,
            pltpu.SemaphoreType.DMA,
            pltpu.SemaphoreType.DMA,
            pltpu.SemaphoreType.DMA,
        ],
    )
    def _k(xfb_hbm, dst_hbm, wpr_hbm, buf_hbm, wgt_hbm,
           idx0, idx1, rows0, rows1, wg0, wg1, s_in, s_wi, s_sc, s_ws):
        wid = lax.axis_index("s") * NC + lax.axis_index("c")
        idx_b = [idx0, idx1]
        rows_b = [rows0, rows1]
        wgt_b = [wg0, wg1]
        in_h = [None] * NCH
        win_h = [None] * NCH
        sc_h = [None] * NCH
        ws_h = [None] * NCH

        def start_in(c):
            pbase = wid * PPW + c * DCH
            src = lax.rem(pbase, N)
            pltpu.sync_copy(dst_hbm.at[pl.ds(pbase, DCH)], idx_b[c % 2])
            in_h[c] = pltpu.async_copy(
                xfb_hbm.at[pl.ds(src, DCH)], rows_b[c % 2], s_in)
            win_h[c] = pltpu.async_copy(
                wpr_hbm.at[pl.ds(pbase, DCH)], wgt_b[c % 2], s_wi)

        start_in(0)
        for c in range(NCH):
            if c + 1 < NCH:
                if c >= 1:
                    sc_h[c - 1].wait()
                    ws_h[c - 1].wait()
                start_in(c + 1)
            in_h[c].wait()
            win_h[c].wait()
            sc_h[c] = pltpu.async_copy(rows_b[c % 2], buf_hbm.at[idx_b[c % 2]],
                                       s_sc)
            ws_h[c] = pltpu.async_copy(wgt_b[c % 2], wgt_hbm.at[idx_b[c % 2]],
                                       s_ws)
        for j in (NCH - 2, NCH - 1):
            sc_h[j].wait()
            ws_h[j].wait()

    return _k(xfb, dst, wpr)


CB = 5
CAP_BLK = CAP // CB


def _ffn_body(cnt_ref, buf_ref, wgt_ref, w1_ref, w3_ref, w2_ref, out_ref):
    e = pl.program_id(0)
    cb = pl.program_id(1)
    cval = cnt_ref[0, e]

    @pl.when(cval > cb * CAP_BLK)
    def _compute():
        rows = lax.broadcasted_iota(jnp.int32, (CAP_BLK, 1), 0) + cb * CAP_BLK
        mask = rows < cval
        wcol = wgt_ref[...][:, 0:1]
        w = buf_ref[...]
        lo = lax.bitcast_convert_type(lax.shift_left(w, 16), jnp.float32)
        hi = lax.bitcast_convert_type(w & jnp.int32(-65536), jnp.float32)
        blk = jnp.concatenate([lo, hi], axis=1)
        a = jnp.where(mask, blk * wcol, 0.0)
        h1 = jnp.dot(a, w1_ref[0], preferred_element_type=jnp.float32,
                     precision=lax.Precision.DEFAULT)
        h3 = jnp.dot(a, w3_ref[0], preferred_element_type=jnp.float32,
                     precision=lax.Precision.DEFAULT)
        g = (h1 * jax.nn.sigmoid(h1)) * h3
        out_ref[...] = jnp.dot(g, w2_ref[0], preferred_element_type=jnp.float32,
                               precision=lax.Precision.DEFAULT)

    @pl.when(cval <= cb * CAP_BLK)
    def _zero():
        out_ref[...] = jnp.zeros((CAP_BLK, H), jnp.float32)


def _ffn(cnt, buf, wgt, w1b, w3b, w2b):
    return pl.pallas_call(
        _ffn_body,
        grid=(E, CB),
        in_specs=[
            pl.BlockSpec(memory_space=pltpu.SMEM),
            pl.BlockSpec((CAP_BLK, H // 2), lambda e, cb: (e * CB + cb, 0)),
            pl.BlockSpec((CAP_BLK, WGT_W), lambda e, cb: (e * CB + cb, 0)),
            pl.BlockSpec((1, H, FFN), lambda e, cb: (e, 0, 0)),
            pl.BlockSpec((1, H, FFN), lambda e, cb: (e, 0, 0)),
            pl.BlockSpec((1, FFN, H), lambda e, cb: (e, 0, 0)),
        ],
        out_specs=pl.BlockSpec((CAP_BLK, H), lambda e, cb: (e * CB + cb, 0)),
        out_shape=jax.ShapeDtypeStruct((R, H), jnp.float32),
    )(cnt, buf, wgt, w1b, w3b, w2b)


TPW = N // NW
CCH = 16


def _combine(eout, gat):
    mesh = plsc.VectorSubcoreMesh(core_axis_name="c", subcore_axis_name="s")
    NCH = TPW // CCH

    @functools.partial(
        pl.kernel,
        out_type=jax.ShapeDtypeStruct((N, H), jnp.float32),
        mesh=mesh,
        scratch_types=[
            pltpu.VMEM((CCH,), jnp.int32),
            pltpu.VMEM((CCH,), jnp.int32),
            pltpu.VMEM((CCH,), jnp.int32),
            pltpu.VMEM((CCH,), jnp.int32),
            pltpu.VMEM((CCH, H), jnp.float32),
            pltpu.VMEM((CCH, H), jnp.float32),
            pltpu.VMEM((CCH, H), jnp.float32),
            pltpu.VMEM((CCH, H), jnp.float32),
            pltpu.SemaphoreType.DMA,
            pltpu.SemaphoreType.DMA,
        ],
    )
    def _k(eout_hbm, gat_hbm, out_hbm, ia0, ia1, ib0, ib1,
           r0a, r0b, r1a, r1b, s_g, s_o):
        wid = lax.axis_index("s") * NC + lax.axis_index("c")
        i0_b = [ia0, ia1]
        i1_b = [ib0, ib1]
        r0_b = [r0a, r0b]
        r1_b = [r1a, r1b]
        g0 = [None] * NCH
        g1 = [None] * NCH
        out_h = [None] * NCH

        def start_g(c):
            tb = wid * TPW + c * CCH
            pltpu.sync_copy(gat_hbm.at[pl.ds(tb, CCH)], i0_b[c % 2])
            pltpu.sync_copy(gat_hbm.at[pl.ds(N + tb, CCH)], i1_b[c % 2])
            g0[c] = pltpu.async_copy(eout_hbm.at[i0_b[c % 2]], r0_b[c % 2], s_g)
            g1[c] = pltpu.async_copy(eout_hbm.at[i1_b[c % 2]], r1_b[c % 2], s_g)

        start_g(0)
        for c in range(NCH):
            if c + 1 < NCH:
                if c >= 1:
                    out_h[c - 1].wait()
                start_g(c + 1)
            g0[c].wait()
            g1[c].wait()
            r0 = r0_b[c % 2]
            r1 = r1_b[c % 2]

            def _row(r, _):
                for h in range(H // 16):
                    sl = pl.ds(h * 16, 16)
                    r0[r, sl] = r0[r, sl] + r1[r, sl]
                return 0

            lax.fori_loop(0, CCH, _row, 0)
            tb = wid * TPW + c * CCH
            out_h[c] = pltpu.async_copy(r0, out_hbm.at[pl.ds(tb, CCH)], s_o)
        for j in (NCH - 2, NCH - 1):
            out_h[j].wait()

    return _k(eout, gat)


def kernel(x, rms_g, Wr, w1, w3, w2):
    xf = x.reshape(N, H)
    g2d = rms_g.reshape(1, H)
    dst, gat, wpr, cnt, aux, xfb = _routing(xf, g2d, Wr)
    buf, wgt = _dispatch(xfb, dst, wpr)
    eout = _ffn(cnt, buf, wgt, w1, w3, w2)
    out = _combine(eout, gat)
    return out.reshape(B, S, H), aux.reshape(())


# speedup vs baseline: 1.0756x; 1.0756x over previous
import functools

import jax
import jax.numpy as jnp
from jax import lax
from jax.experimental import pallas as pl
from jax.experimental.pallas import tpu as pltpu
from jax.experimental.pallas import tpu_sc as plsc

H = 1024
E = 8
K = 2
B = 2
S = 2048
N = B * S
CAP = int(1.25 * N * K / E)
ECAP = E * CAP
R = ECAP + 8
FFN = 1536
AUX = 0.01
P = N * K
WGT_W = 128

NC = 2
NS = 16
NW = NC * NS

TOK_BLK = 256
NB = N // TOK_BLK


def _routing_body(x_ref, g_ref, wr_ref, dst_ref, gat_ref, wpr_ref, cnt_ref,
                  aux_ref, xfb_ref, carry, mesum):
    i = pl.program_id(0)

    @pl.when(i == 0)
    def _init():
        carry[...] = jnp.zeros((1, E), jnp.float32)
        mesum[...] = jnp.zeros((1, E), jnp.float32)

    @pl.when(i < 2 * NB)
    def _main():
        kk = i // NB
        b = i % NB
        xb = x_ref[...]
        ms = jnp.mean(xb * xb, axis=1, keepdims=True)
        xn = xb * lax.rsqrt(ms + 1e-6) * g_ref[...]
        logits = jnp.dot(xn, wr_ref[...], preferred_element_type=jnp.float32)
        mx = jnp.max(logits, axis=1, keepdims=True)
        ex = jnp.exp(logits - mx)
        p = ex / jnp.sum(ex, axis=1, keepdims=True)
        iota8 = lax.broadcasted_iota(jnp.int32, (TOK_BLK, E), 1)
        m1 = jnp.max(p, axis=1, keepdims=True)
        i1 = jnp.min(jnp.where(p == m1, iota8, E), axis=1, keepdims=True)
        p2 = jnp.where(iota8 == i1, -jnp.inf, p)
        m2 = jnp.max(p2, axis=1, keepdims=True)
        i2 = jnp.min(jnp.where(p2 == m2, iota8, E), axis=1, keepdims=True)
        tot = m1 + m2
        wk = jnp.where(kk == 0, m1 / tot, m2 / tot)
        ek = jnp.where(kk == 0, i1, i2)
        oh = (iota8 == ek).astype(jnp.float32)
        r_io = lax.broadcasted_iota(jnp.int32, (TOK_BLK, TOK_BLK), 0)
        c_io = lax.broadcasted_iota(jnp.int32, (TOK_BLK, TOK_BLK), 1)
        tri = (r_io >= c_io).astype(jnp.float32)
        csum = jnp.dot(tri, oh, preferred_element_type=jnp.float32,
                       precision=lax.Precision.HIGHEST)
        base = carry[...]
        posm = csum - 1.0 + base
        pos = jnp.sum(oh * posm, axis=1, keepdims=True).astype(jnp.int32)
        assigned = pos < CAP
        slot = ek * CAP + pos
        off = kk * N + b * TOK_BLK
        dst_ref[pl.ds(off, TOK_BLK)] = jnp.where(assigned, slot, ECAP).reshape(TOK_BLK)
        gat_ref[pl.ds(off, TOK_BLK)] = jnp.where(assigned, slot, -1).reshape(TOK_BLK)
        wpr_ref[pl.ds(off, TOK_BLK), :] = jnp.broadcast_to(wk, (TOK_BLK, WGT_W))
        carry[...] = base + jnp.sum(oh, axis=0, keepdims=True)

        @pl.when(kk == 0)
        def _xb():
            xr = xb.astype(jnp.bfloat16).astype(jnp.float32)
            u = lax.bitcast_convert_type(xr, jnp.int32)
            lo = lax.shift_right_logical(u[:, :H // 2], 16)
            hi = u[:, H // 2:] & jnp.int32(-65536)
            xfb_ref[pl.ds(b * TOK_BLK, TOK_BLK), :] = lo | hi

        @pl.when(kk == 0)
        def _me():
            mesum[...] = mesum[...] + jnp.sum(p, axis=0, keepdims=True)

    @pl.when(i == 2 * NB)
    def _final():
        ctot = carry[...]
        cc = jnp.minimum(ctot, float(CAP))
        cnt_ref[...] = cc.astype(jnp.int32)
        mn = jnp.min(cc)
        e_io = lax.broadcasted_iota(jnp.int32, (1, E), 1)
        e0 = jnp.min(jnp.where(cc == mn, e_io, E))
        zslot = e0 * CAP + mn.astype(jnp.int32)
        g_all = gat_ref[...]
        gat_ref[...] = jnp.where(g_all < 0, zslot, g_all)
        me = mesum[...] / N
        ce = ctot / (N * K)
        aux_ref[...] = AUX * E * jnp.sum(me * ce, axis=1, keepdims=True)


def _routing(xf, g2d, Wr):
    grid = (2 * NB + 1,)
    return pl.pallas_call(
        _routing_body,
        grid=grid,
        in_specs=[
            pl.BlockSpec((TOK_BLK, H), lambda i: (i % NB, 0)),
            pl.BlockSpec((1, H), lambda i: (0, 0)),
            pl.BlockSpec((H, E), lambda i: (0, 0)),
        ],
        out_specs=[
            pl.BlockSpec((P,), lambda i: (0,)),
            pl.BlockSpec((P,), lambda i: (0,)),
            pl.BlockSpec((P, WGT_W), lambda i: (0, 0)),
            pl.BlockSpec((1, E), lambda i: (0, 0)),
            pl.BlockSpec((1, 1), lambda i: (0, 0)),
            pl.BlockSpec((N, H // 2), lambda i: (0, 0)),
        ],
        out_shape=[
            jax.ShapeDtypeStruct((P,), jnp.int32),
            jax.ShapeDtypeStruct((P,), jnp.int32),
            jax.ShapeDtypeStruct((P, WGT_W), jnp.float32),
            jax.ShapeDtypeStruct((1, E), jnp.int32),
            jax.ShapeDtypeStruct((1, 1), jnp.float32),
            jax.ShapeDtypeStruct((N, H // 2), jnp.int32),
        ],
        scratch_shapes=[
            pltpu.VMEM((1, E), jnp.float32),
            pltpu.VMEM((1, E), jnp.float32),
        ],
    )(xf, g2d, Wr)


PPW = P // NW
DCH = 64


def _dispatch(xfb, dst, wpr):
    mesh = plsc.VectorSubcoreMesh(core_axis_name="c", subcore_axis_name="s")
    NCH = PPW // DCH

    @functools.partial(
        pl.kernel,
        out_type=(jax.ShapeDtypeStruct((R, H // 2), jnp.int32),
                  jax.ShapeDtypeStruct((R, WGT_W), jnp.float32)),
        mesh=mesh,
        scratch_types=[
            pltpu.VMEM((DCH,), jnp.int32),
            pltpu.VMEM((DCH,), jnp.int32),
            pltpu.VMEM((DCH, H // 2), jnp.int32),
            pltpu.VMEM((DCH, H // 2), jnp.int32),
            pltpu.VMEM((DCH, WGT_W), jnp.float32),
            pltpu.VMEM((DCH, WGT_W), jnp.float32),
            pltpu.SemaphoreType.DMA,
            pltpu.SemaphoreType.DMA,
            pltpu.SemaphoreType.DMA,
            pltpu.SemaphoreType.DMA,
        ],
    )
    def _k(xfb_hbm, dst_hbm, wpr_hbm, buf_hbm, wgt_hbm,
           idx0, idx1, rows0, rows1, wg0, wg1, s_in, s_wi, s_sc, s_ws):
        wid = lax.axis_index("s") * NC + lax.axis_index("c")
        idx_b = [idx0, idx1]
        rows_b = [rows0, rows1]
        wgt_b = [wg0, wg1]
        in_h = [None] * NCH
        win_h = [None] * NCH
        sc_h = [None] * NCH
        ws_h = [None] * NCH

        def start_in(c):
            pbase = wid * PPW + c * DCH
            src = lax.rem(pbase, N)
            pltpu.sync_copy(dst_hbm.at[pl.ds(pbase, DCH)], idx_b[c % 2])
            in_h[c] = pltpu.async_copy(
                xfb_hbm.at[pl.ds(src, DCH)], rows_b[c % 2], s_in)
            win_h[c] = pltpu.async_copy(
                wpr_hbm.at[pl.ds(pbase, DCH)], wgt_b[c % 2], s_wi)

        start_in(0)
        for c in range(NCH):
            if c + 1 < NCH:
                if c >= 1:
                    sc_h[c - 1].wait()
                    ws_h[c - 1].wait()
                start_in(c + 1)
            in_h[c].wait()
            win_h[c].wait()
            sc_h[c] = pltpu.async_copy(rows_b[c % 2], buf_hbm.at[idx_b[c % 2]],
                                       s_sc)
            ws_h[c] = pltpu.async_copy(wgt_b[c % 2], wgt_hbm.at[idx_b[c % 2]],
                                       s_ws)
        for j in (NCH - 2, NCH - 1):
            sc_h[j].wait()
            ws_h[j].wait()

    return _k(xfb, dst, wpr)


CB = 2
CAP_BLK = CAP // CB


def _ffn_body(cnt_ref, buf_ref, wgt_ref, w1_ref, w3_ref, w2_ref, out_ref):
    e = pl.program_id(0)
    cb = pl.program_id(1)
    cval = cnt_ref[0, e]
    rows = lax.broadcasted_iota(jnp.int32, (CAP_BLK, 1), 0) + cb * CAP_BLK
    mask = rows < cval
    wcol = wgt_ref[...][:, 0:1]
    w = buf_ref[...]
    lo = lax.bitcast_convert_type(lax.shift_left(w, 16), jnp.float32)
    hi = lax.bitcast_convert_type(w & jnp.int32(-65536), jnp.float32)
    blk = jnp.concatenate([lo, hi], axis=1)
    a = jnp.where(mask, blk * wcol, 0.0)
    h1 = jnp.dot(a, w1_ref[0], preferred_element_type=jnp.float32,
                 precision=lax.Precision.DEFAULT)
    h3 = jnp.dot(a, w3_ref[0], preferred_element_type=jnp.float32,
                 precision=lax.Precision.DEFAULT)
    g = (h1 * jax.nn.sigmoid(h1)) * h3
    out_ref[...] = jnp.dot(g, w2_ref[0], preferred_element_type=jnp.float32,
                           precision=lax.Precision.DEFAULT)


def _ffn(cnt, buf, wgt, w1b, w3b, w2b):
    return pl.pallas_call(
        _ffn_body,
        grid=(E, CB),
        in_specs=[
            pl.BlockSpec(memory_space=pltpu.SMEM),
            pl.BlockSpec((CAP_BLK, H // 2), lambda e, cb: (e * CB + cb, 0)),
            pl.BlockSpec((CAP_BLK, WGT_W), lambda e, cb: (e * CB + cb, 0)),
            pl.BlockSpec((1, H, FFN), lambda e, cb: (e, 0, 0)),
            pl.BlockSpec((1, H, FFN), lambda e, cb: (e, 0, 0)),
            pl.BlockSpec((1, FFN, H), lambda e, cb: (e, 0, 0)),
        ],
        out_specs=pl.BlockSpec((CAP_BLK, H), lambda e, cb: (e * CB + cb, 0)),
        out_shape=jax.ShapeDtypeStruct((R, H), jnp.float32),
    )(cnt, buf, wgt, w1b, w3b, w2b)


TPW = N // NW
CCH = 16


def _combine(eout, gat):
    mesh = plsc.VectorSubcoreMesh(core_axis_name="c", subcore_axis_name="s")
    NCH = TPW // CCH

    @functools.partial(
        pl.kernel,
        out_type=jax.ShapeDtypeStruct((N, H), jnp.float32),
        mesh=mesh,
        scratch_types=[
            pltpu.VMEM((CCH,), jnp.int32),
            pltpu.VMEM((CCH,), jnp.int32),
            pltpu.VMEM((CCH,), jnp.int32),
            pltpu.VMEM((CCH,), jnp.int32),
            pltpu.VMEM((CCH, H), jnp.float32),
            pltpu.VMEM((CCH, H), jnp.float32),
            pltpu.VMEM((CCH, H), jnp.float32),
            pltpu.VMEM((CCH, H), jnp.float32),
            pltpu.SemaphoreType.DMA,
            pltpu.SemaphoreType.DMA,
        ],
    )
    def _k(eout_hbm, gat_hbm, out_hbm, ia0, ia1, ib0, ib1,
           r0a, r0b, r1a, r1b, s_g, s_o):
        wid = lax.axis_index("s") * NC + lax.axis_index("c")
        i0_b = [ia0, ia1]
        i1_b = [ib0, ib1]
        r0_b = [r0a, r0b]
        r1_b = [r1a, r1b]
        g0 = [None] * NCH
        g1 = [None] * NCH
        out_h = [None] * NCH

        def start_g(c):
            tb = wid * TPW + c * CCH
            pltpu.sync_copy(gat_hbm.at[pl.ds(tb, CCH)], i0_b[c % 2])
            pltpu.sync_copy(gat_hbm.at[pl.ds(N + tb, CCH)], i1_b[c % 2])
            g0[c] = pltpu.async_copy(eout_hbm.at[i0_b[c % 2]], r0_b[c % 2], s_g)
            g1[c] = pltpu.async_copy(eout_hbm.at[i1_b[c % 2]], r1_b[c % 2], s_g)

        start_g(0)
        for c in range(NCH):
            if c + 1 < NCH:
                if c >= 1:
                    out_h[c - 1].wait()
                start_g(c + 1)
            g0[c].wait()
            g1[c].wait()
            r0 = r0_b[c % 2]
            r1 = r1_b[c % 2]

            def _row(r, _):
                for h in range(H // 16):
                    sl = pl.ds(h * 16, 16)
                    r0[r, sl] = r0[r, sl] + r1[r, sl]
                return 0

            lax.fori_loop(0, CCH, _row, 0)
            tb = wid * TPW + c * CCH
            out_h[c] = pltpu.async_copy(r0, out_hbm.at[pl.ds(tb, CCH)], s_o)
        for j in (NCH - 2, NCH - 1):
            out_h[j].wait()

    return _k(eout, gat)


def kernel(x, rms_g, Wr, w1, w3, w2):
    xf = x.reshape(N, H)
    g2d = rms_g.reshape(1, H)
    dst, gat, wpr, cnt, aux, xfb = _routing(xf, g2d, Wr)
    buf, wgt = _dispatch(xfb, dst, wpr)
    eout = _ffn(cnt, buf, wgt, w1, w3, w2)
    out = _combine(eout, gat)
    return out.reshape(B, S, H), aux.reshape(())


# speedup vs baseline: 1.1219x; 1.0430x over previous
import functools

import jax
import jax.numpy as jnp
from jax import lax
from jax.experimental import pallas as pl
from jax.experimental.pallas import tpu as pltpu
from jax.experimental.pallas import tpu_sc as plsc

H = 1024
E = 8
K = 2
B = 2
S = 2048
N = B * S
CAP = int(1.25 * N * K / E)
ECAP = E * CAP
R = ECAP + 8
FFN = 1536
AUX = 0.01
P = N * K
WGT_W = 128

NC = 2
NS = 16
NW = NC * NS

TOK_BLK = 256
NB = N // TOK_BLK


def _routing_body(x_ref, g_ref, wr_ref, dst_ref, gat_ref, wpr_ref, cnt_ref,
                  aux_ref, xfb_ref, carry, mesum):
    i = pl.program_id(0)

    @pl.when(i == 0)
    def _init():
        carry[...] = jnp.zeros((1, E), jnp.float32)
        mesum[...] = jnp.zeros((1, E), jnp.float32)

    @pl.when(i < 2 * NB)
    def _main():
        kk = i // NB
        b = i % NB
        xb = x_ref[...]
        ms = jnp.mean(xb * xb, axis=1, keepdims=True)
        xn = xb * lax.rsqrt(ms + 1e-6) * g_ref[...]
        logits = jnp.dot(xn, wr_ref[...], preferred_element_type=jnp.float32)
        mx = jnp.max(logits, axis=1, keepdims=True)
        ex = jnp.exp(logits - mx)
        p = ex / jnp.sum(ex, axis=1, keepdims=True)
        iota8 = lax.broadcasted_iota(jnp.int32, (TOK_BLK, E), 1)
        m1 = jnp.max(p, axis=1, keepdims=True)
        i1 = jnp.min(jnp.where(p == m1, iota8, E), axis=1, keepdims=True)
        p2 = jnp.where(iota8 == i1, -jnp.inf, p)
        m2 = jnp.max(p2, axis=1, keepdims=True)
        i2 = jnp.min(jnp.where(p2 == m2, iota8, E), axis=1, keepdims=True)
        tot = m1 + m2
        wk = jnp.where(kk == 0, m1 / tot, m2 / tot)
        ek = jnp.where(kk == 0, i1, i2)
        oh = (iota8 == ek).astype(jnp.float32)
        r_io = lax.broadcasted_iota(jnp.int32, (TOK_BLK, TOK_BLK), 0)
        c_io = lax.broadcasted_iota(jnp.int32, (TOK_BLK, TOK_BLK), 1)
        tri = (r_io >= c_io).astype(jnp.float32)
        csum = jnp.dot(tri, oh, preferred_element_type=jnp.float32)
        base = carry[...]
        posm = csum - 1.0 + base
        pos = jnp.sum(oh * posm, axis=1, keepdims=True).astype(jnp.int32)
        assigned = pos < CAP
        slot = ek * CAP + pos
        off = kk * N + b * TOK_BLK
        dst_ref[pl.ds(off, TOK_BLK)] = jnp.where(assigned, slot, ECAP).reshape(TOK_BLK)
        gat_ref[pl.ds(off, TOK_BLK)] = jnp.where(assigned, slot, -1).reshape(TOK_BLK)
        wpr_ref[pl.ds(off, TOK_BLK), :] = jnp.broadcast_to(wk, (TOK_BLK, WGT_W))
        carry[...] = base + jnp.sum(oh, axis=0, keepdims=True)

        @pl.when(kk == 0)
        def _xb():
            xr = xb.astype(jnp.bfloat16).astype(jnp.float32)
            u = lax.bitcast_convert_type(xr, jnp.int32)
            lo = lax.shift_right_logical(u[:, :H // 2], 16)
            hi = u[:, H // 2:] & jnp.int32(-65536)
            xfb_ref[pl.ds(b * TOK_BLK, TOK_BLK), :] = lo | hi

        @pl.when(kk == 0)
        def _me():
            mesum[...] = mesum[...] + jnp.sum(p, axis=0, keepdims=True)

    @pl.when(i == 2 * NB)
    def _final():
        ctot = carry[...]
        cc = jnp.minimum(ctot, float(CAP))
        cnt_ref[...] = cc.astype(jnp.int32)
        mn = jnp.min(cc)
        e_io = lax.broadcasted_iota(jnp.int32, (1, E), 1)
        e0 = jnp.min(jnp.where(cc == mn, e_io, E))
        zslot = e0 * CAP + mn.astype(jnp.int32)
        g_all = gat_ref[...]
        gat_ref[...] = jnp.where(g_all < 0, zslot, g_all)
        me = mesum[...] / N
        ce = ctot / (N * K)
        aux_ref[...] = AUX * E * jnp.sum(me * ce, axis=1, keepdims=True)


def _routing(xf, g2d, Wr):
    grid = (2 * NB + 1,)
    return pl.pallas_call(
        _routing_body,
        grid=grid,
        in_specs=[
            pl.BlockSpec((TOK_BLK, H), lambda i: (i % NB, 0)),
            pl.BlockSpec((1, H), lambda i: (0, 0)),
            pl.BlockSpec((H, E), lambda i: (0, 0)),
        ],
        out_specs=[
            pl.BlockSpec((P,), lambda i: (0,)),
            pl.BlockSpec((P,), lambda i: (0,)),
            pl.BlockSpec((P, WGT_W), lambda i: (0, 0)),
            pl.BlockSpec((1, E), lambda i: (0, 0)),
            pl.BlockSpec((1, 1), lambda i: (0, 0)),
            pl.BlockSpec((N, H // 2), lambda i: (0, 0)),
        ],
        out_shape=[
            jax.ShapeDtypeStruct((P,), jnp.int32),
            jax.ShapeDtypeStruct((P,), jnp.int32),
            jax.ShapeDtypeStruct((P, WGT_W), jnp.float32),
            jax.ShapeDtypeStruct((1, E), jnp.int32),
            jax.ShapeDtypeStruct((1, 1), jnp.float32),
            jax.ShapeDtypeStruct((N, H // 2), jnp.int32),
        ],
        scratch_shapes=[
            pltpu.VMEM((1, E), jnp.float32),
            pltpu.VMEM((1, E), jnp.float32),
        ],
    )(xf, g2d, Wr)


PPW = P // NW
DCH = 64


def _dispatch(xfb, dst, wpr):
    mesh = plsc.VectorSubcoreMesh(core_axis_name="c", subcore_axis_name="s")
    NCH = PPW // DCH

    @functools.partial(
        pl.kernel,
        out_type=(jax.ShapeDtypeStruct((R, H // 2), jnp.int32),
                  jax.ShapeDtypeStruct((R, WGT_W), jnp.float32)),
        mesh=mesh,
        scratch_types=[
            pltpu.VMEM((DCH,), jnp.int32),
            pltpu.VMEM((DCH,), jnp.int32),
            pltpu.VMEM((DCH, H // 2), jnp.int32),
            pltpu.VMEM((DCH, H // 2), jnp.int32),
            pltpu.VMEM((DCH, WGT_W), jnp.float32),
            pltpu.VMEM((DCH, WGT_W), jnp.float32),
            pltpu.SemaphoreType.DMA,
            pltpu.SemaphoreType.DMA,
            pltpu.SemaphoreType.DMA,
            pltpu.SemaphoreType.DMA,
        ],
    )
    def _k(xfb_hbm, dst_hbm, wpr_hbm, buf_hbm, wgt_hbm,
           idx0, idx1, rows0, rows1, wg0, wg1, s_in, s_wi, s_sc, s_ws):
        wid = lax.axis_index("s") * NC + lax.axis_index("c")
        idx_b = [idx0, idx1]
        rows_b = [rows0, rows1]
        wgt_b = [wg0, wg1]
        in_h = [None] * NCH
        win_h = [None] * NCH
        sc_h = [None] * NCH
        ws_h = [None] * NCH

        def start_in(c):
            pbase = wid * PPW + c * DCH
            src = lax.rem(pbase, N)
            pltpu.sync_copy(dst_hbm.at[pl.ds(pbase, DCH)], idx_b[c % 2])
            in_h[c] = pltpu.async_copy(
                xfb_hbm.at[pl.ds(src, DCH)], rows_b[c % 2], s_in)
            win_h[c] = pltpu.async_copy(
                wpr_hbm.at[pl.ds(pbase, DCH)], wgt_b[c % 2], s_wi)

        start_in(0)
        for c in range(NCH):
            if c + 1 < NCH:
                if c >= 1:
                    sc_h[c - 1].wait()
                    ws_h[c - 1].wait()
                start_in(c + 1)
            in_h[c].wait()
            win_h[c].wait()
            sc_h[c] = pltpu.async_copy(rows_b[c % 2], buf_hbm.at[idx_b[c % 2]],
                                       s_sc)
            ws_h[c] = pltpu.async_copy(wgt_b[c % 2], wgt_hbm.at[idx_b[c % 2]],
                                       s_ws)
        for j in (NCH - 2, NCH - 1):
            sc_h[j].wait()
            ws_h[j].wait()

    return _k(xfb, dst, wpr)


CB = 2
CAP_BLK = CAP // CB


def _ffn_body(cnt_ref, buf_ref, wgt_ref, w1_ref, w3_ref, w2_ref, out_ref):
    e = pl.program_id(0)
    cb = pl.program_id(1)
    cval = cnt_ref[0, e]
    rows = lax.broadcasted_iota(jnp.int32, (CAP_BLK, 1), 0) + cb * CAP_BLK
    mask = rows < cval
    wcol = wgt_ref[...][:, 0:1]
    w = buf_ref[...]
    lo = lax.bitcast_convert_type(lax.shift_left(w, 16), jnp.float32)
    hi = lax.bitcast_convert_type(w & jnp.int32(-65536), jnp.float32)
    blk = jnp.concatenate([lo, hi], axis=1)
    a = jnp.where(mask, blk * wcol, 0.0)
    h1 = jnp.dot(a, w1_ref[0], preferred_element_type=jnp.float32,
                 precision=lax.Precision.DEFAULT)
    h3 = jnp.dot(a, w3_ref[0], preferred_element_type=jnp.float32,
                 precision=lax.Precision.DEFAULT)
    g = (h1 * jax.nn.sigmoid(h1)) * h3
    out_ref[...] = jnp.dot(g, w2_ref[0], preferred_element_type=jnp.float32,
                           precision=lax.Precision.DEFAULT)


def _ffn(cnt, buf, wgt, w1b, w3b, w2b):
    return pl.pallas_call(
        _ffn_body,
        grid=(E, CB),
        in_specs=[
            pl.BlockSpec(memory_space=pltpu.SMEM),
            pl.BlockSpec((CAP_BLK, H // 2), lambda e, cb: (e * CB + cb, 0)),
            pl.BlockSpec((CAP_BLK, WGT_W), lambda e, cb: (e * CB + cb, 0)),
            pl.BlockSpec((1, H, FFN), lambda e, cb: (e, 0, 0)),
            pl.BlockSpec((1, H, FFN), lambda e, cb: (e, 0, 0)),
            pl.BlockSpec((1, FFN, H), lambda e, cb: (e, 0, 0)),
        ],
        out_specs=pl.BlockSpec((CAP_BLK, H), lambda e, cb: (e * CB + cb, 0)),
        out_shape=jax.ShapeDtypeStruct((R, H), jnp.float32),
    )(cnt, buf, wgt, w1b, w3b, w2b)


TPW = N // NW
CCH = 16


def _combine(eout, gat):
    mesh = plsc.VectorSubcoreMesh(core_axis_name="c", subcore_axis_name="s")
    NCH = TPW // CCH

    @functools.partial(
        pl.kernel,
        out_type=jax.ShapeDtypeStruct((N, H), jnp.float32),
        mesh=mesh,
        scratch_types=[
            pltpu.VMEM((CCH,), jnp.int32),
            pltpu.VMEM((CCH,), jnp.int32),
            pltpu.VMEM((CCH,), jnp.int32),
            pltpu.VMEM((CCH,), jnp.int32),
            pltpu.VMEM((CCH, H), jnp.float32),
            pltpu.VMEM((CCH, H), jnp.float32),
            pltpu.VMEM((CCH, H), jnp.float32),
            pltpu.VMEM((CCH, H), jnp.float32),
            pltpu.SemaphoreType.DMA,
            pltpu.SemaphoreType.DMA,
        ],
    )
    def _k(eout_hbm, gat_hbm, out_hbm, ia0, ia1, ib0, ib1,
           r0a, r0b, r1a, r1b, s_g, s_o):
        wid = lax.axis_index("s") * NC + lax.axis_index("c")
        i0_b = [ia0, ia1]
        i1_b = [ib0, ib1]
        r0_b = [r0a, r0b]
        r1_b = [r1a, r1b]
        g0 = [None] * NCH
        g1 = [None] * NCH
        out_h = [None] * NCH

        def start_g(c):
            tb = wid * TPW + c * CCH
            pltpu.sync_copy(gat_hbm.at[pl.ds(tb, CCH)], i0_b[c % 2])
            pltpu.sync_copy(gat_hbm.at[pl.ds(N + tb, CCH)], i1_b[c % 2])
            g0[c] = pltpu.async_copy(eout_hbm.at[i0_b[c % 2]], r0_b[c % 2], s_g)
            g1[c] = pltpu.async_copy(eout_hbm.at[i1_b[c % 2]], r1_b[c % 2], s_g)

        start_g(0)
        for c in range(NCH):
            if c + 1 < NCH:
                if c >= 1:
                    out_h[c - 1].wait()
                start_g(c + 1)
            g0[c].wait()
            g1[c].wait()
            r0 = r0_b[c % 2]
            r1 = r1_b[c % 2]

            def _row(r, _):
                for h in range(H // 16):
                    sl = pl.ds(h * 16, 16)
                    r0[r, sl] = r0[r, sl] + r1[r, sl]
                return 0

            lax.fori_loop(0, CCH, _row, 0)
            tb = wid * TPW + c * CCH
            out_h[c] = pltpu.async_copy(r0, out_hbm.at[pl.ds(tb, CCH)], s_o)
        for j in (NCH - 2, NCH - 1):
            out_h[j].wait()

    return _k(eout, gat)


def kernel(x, rms_g, Wr, w1, w3, w2):
    xf = x.reshape(N, H)
    g2d = rms_g.reshape(1, H)
    dst, gat, wpr, cnt, aux, xfb = _routing(xf, g2d, Wr)
    buf, wgt = _dispatch(xfb, dst, wpr)
    eout = _ffn(cnt, buf, wgt, w1, w3, w2)
    out = _combine(eout, gat)
    return out.reshape(B, S, H), aux.reshape(())


# speedup vs baseline: 1.2275x; 1.0941x over previous
import functools

import jax
import jax.numpy as jnp
from jax import lax
from jax.experimental import pallas as pl
from jax.experimental.pallas import tpu as pltpu
from jax.experimental.pallas import tpu_sc as plsc

H = 1024
E = 8
K = 2
B = 2
S = 2048
N = B * S
CAP = int(1.25 * N * K / E)
ECAP = E * CAP
R = ECAP + 8
FFN = 1536
AUX = 0.01
P = N * K
WGT_W = 128

NC = 2
NS = 16
NW = NC * NS

TOK_BLK = 256
NB = N // TOK_BLK


def _routing_body(x_ref, g_ref, wr_ref, dst_ref, gat_ref, wpr_ref, cnt_ref,
                  aux_ref, xfb_ref, carry, mesum, oh_s):
    i = pl.program_id(0)

    @pl.when(i == 0)
    def _init():
        carry[...] = jnp.zeros((1, E), jnp.float32)
        mesum[...] = jnp.zeros((1, E), jnp.float32)

    @pl.when(i < NB)
    def _pass1():
        b = i
        xb = x_ref[...]
        ms = jnp.mean(xb * xb, axis=1, keepdims=True)
        xn = xb * lax.rsqrt(ms + 1e-6) * g_ref[...]
        logits = jnp.dot(xn, wr_ref[...], preferred_element_type=jnp.float32)
        mx = jnp.max(logits, axis=1, keepdims=True)
        ex = jnp.exp(logits - mx)
        p = ex / jnp.sum(ex, axis=1, keepdims=True)
        iota8 = lax.broadcasted_iota(jnp.int32, (TOK_BLK, E), 1)
        m1 = jnp.max(p, axis=1, keepdims=True)
        i1 = jnp.min(jnp.where(p == m1, iota8, E), axis=1, keepdims=True)
        p2 = jnp.where(iota8 == i1, -jnp.inf, p)
        m2 = jnp.max(p2, axis=1, keepdims=True)
        i2 = jnp.min(jnp.where(p2 == m2, iota8, E), axis=1, keepdims=True)
        tot = m1 + m2
        oh = (iota8 == i1).astype(jnp.float32)
        oh_s[pl.ds(b, 1)] = (iota8 == i2).astype(jnp.float32).reshape(
            1, TOK_BLK, E)
        r_io = lax.broadcasted_iota(jnp.int32, (TOK_BLK, TOK_BLK), 0)
        c_io = lax.broadcasted_iota(jnp.int32, (TOK_BLK, TOK_BLK), 1)
        tri = (r_io >= c_io).astype(jnp.float32)
        csum = jnp.dot(tri, oh, preferred_element_type=jnp.float32)
        base = carry[...]
        posm = csum - 1.0 + base
        pos = jnp.sum(oh * posm, axis=1, keepdims=True).astype(jnp.int32)
        assigned = pos < CAP
        slot = i1 * CAP + pos
        off = b * TOK_BLK
        dst_ref[pl.ds(off, TOK_BLK)] = jnp.where(assigned, slot, ECAP).reshape(TOK_BLK)
        gat_ref[pl.ds(off, TOK_BLK)] = jnp.where(assigned, slot, -1).reshape(TOK_BLK)
        wpr_ref[pl.ds(off, TOK_BLK), :] = jnp.broadcast_to(
            m1 / tot, (TOK_BLK, WGT_W))
        wpr_ref[pl.ds(N + off, TOK_BLK), :] = jnp.broadcast_to(
            m2 / tot, (TOK_BLK, WGT_W))
        carry[...] = base + jnp.sum(oh, axis=0, keepdims=True)
        mesum[...] = mesum[...] + jnp.sum(p, axis=0, keepdims=True)
        xr = xb.astype(jnp.bfloat16).astype(jnp.float32)
        u = lax.bitcast_convert_type(xr, jnp.int32)
        lo = lax.shift_right_logical(u[:, :H // 2], 16)
        hi = u[:, H // 2:] & jnp.int32(-65536)
        xfb_ref[pl.ds(off, TOK_BLK), :] = lo | hi

    @pl.when(jnp.logical_and(i >= NB, i < 2 * NB))
    def _pass2():
        b = i - NB
        oh = oh_s[pl.ds(b, 1)].reshape(TOK_BLK, E)
        iota8 = lax.broadcasted_iota(jnp.int32, (TOK_BLK, E), 1)
        ek = jnp.sum(oh * iota8.astype(jnp.float32), axis=1,
                     keepdims=True).astype(jnp.int32)
        r_io = lax.broadcasted_iota(jnp.int32, (TOK_BLK, TOK_BLK), 0)
        c_io = lax.broadcasted_iota(jnp.int32, (TOK_BLK, TOK_BLK), 1)
        tri = (r_io >= c_io).astype(jnp.float32)
        csum = jnp.dot(tri, oh, preferred_element_type=jnp.float32)
        base = carry[...]
        posm = csum - 1.0 + base
        pos = jnp.sum(oh * posm, axis=1, keepdims=True).astype(jnp.int32)
        assigned = pos < CAP
        slot = ek * CAP + pos
        off = N + b * TOK_BLK
        dst_ref[pl.ds(off, TOK_BLK)] = jnp.where(assigned, slot, ECAP).reshape(TOK_BLK)
        gat_ref[pl.ds(off, TOK_BLK)] = jnp.where(assigned, slot, -1).reshape(TOK_BLK)
        carry[...] = base + jnp.sum(oh, axis=0, keepdims=True)

    @pl.when(i == 2 * NB)
    def _final():
        ctot = carry[...]
        cc = jnp.minimum(ctot, float(CAP))
        cnt_ref[...] = cc.astype(jnp.int32)
        mn = jnp.min(cc)
        e_io = lax.broadcasted_iota(jnp.int32, (1, E), 1)
        e0 = jnp.min(jnp.where(cc == mn, e_io, E))
        zslot = e0 * CAP + mn.astype(jnp.int32)
        g_all = gat_ref[...]
        gat_ref[...] = jnp.where(g_all < 0, zslot, g_all)
        me = mesum[...] / N
        ce = ctot / (N * K)
        aux_ref[...] = AUX * E * jnp.sum(me * ce, axis=1, keepdims=True)


def _routing(xf, g2d, Wr):
    grid = (2 * NB + 1,)
    return pl.pallas_call(
        _routing_body,
        grid=grid,
        in_specs=[
            pl.BlockSpec((TOK_BLK, H), lambda i: (jnp.minimum(i, NB - 1), 0)),
            pl.BlockSpec((1, H), lambda i: (0, 0)),
            pl.BlockSpec((H, E), lambda i: (0, 0)),
        ],
        out_specs=[
            pl.BlockSpec((P,), lambda i: (0,)),
            pl.BlockSpec((P,), lambda i: (0,)),
            pl.BlockSpec((P, WGT_W), lambda i: (0, 0)),
            pl.BlockSpec((1, E), lambda i: (0, 0)),
            pl.BlockSpec((1, 1), lambda i: (0, 0)),
            pl.BlockSpec((N, H // 2), lambda i: (0, 0)),
        ],
        out_shape=[
            jax.ShapeDtypeStruct((P,), jnp.int32),
            jax.ShapeDtypeStruct((P,), jnp.int32),
            jax.ShapeDtypeStruct((P, WGT_W), jnp.float32),
            jax.ShapeDtypeStruct((1, E), jnp.int32),
            jax.ShapeDtypeStruct((1, 1), jnp.float32),
            jax.ShapeDtypeStruct((N, H // 2), jnp.int32),
        ],
        scratch_shapes=[
            pltpu.VMEM((1, E), jnp.float32),
            pltpu.VMEM((1, E), jnp.float32),
            pltpu.VMEM((NB, TOK_BLK, E), jnp.float32),
        ],
    )(xf, g2d, Wr)


PPW = P // NW
DCH = 64


def _dispatch(xfb, dst, wpr):
    mesh = plsc.VectorSubcoreMesh(core_axis_name="c", subcore_axis_name="s")
    NCH = PPW // DCH

    @functools.partial(
        pl.kernel,
        out_type=(jax.ShapeDtypeStruct((R, H // 2), jnp.int32),
                  jax.ShapeDtypeStruct((R, WGT_W), jnp.float32)),
        mesh=mesh,
        scratch_types=[
            pltpu.VMEM((DCH,), jnp.int32),
            pltpu.VMEM((DCH,), jnp.int32),
            pltpu.VMEM((DCH, H // 2), jnp.int32),
            pltpu.VMEM((DCH, H // 2), jnp.int32),
            pltpu.VMEM((DCH, WGT_W), jnp.float32),
            pltpu.VMEM((DCH, WGT_W), jnp.float32),
            pltpu.SemaphoreType.DMA,
            pltpu.SemaphoreType.DMA,
            pltpu.SemaphoreType.DMA,
            pltpu.SemaphoreType.DMA,
        ],
    )
    def _k(xfb_hbm, dst_hbm, wpr_hbm, buf_hbm, wgt_hbm,
           idx0, idx1, rows0, rows1, wg0, wg1, s_in, s_wi, s_sc, s_ws):
        wid = lax.axis_index("s") * NC + lax.axis_index("c")
        idx_b = [idx0, idx1]
        rows_b = [rows0, rows1]
        wgt_b = [wg0, wg1]
        in_h = [None] * NCH
        win_h = [None] * NCH
        sc_h = [None] * NCH
        ws_h = [None] * NCH

        def start_in(c):
            pbase = wid * PPW + c * DCH
            src = lax.rem(pbase, N)
            pltpu.sync_copy(dst_hbm.at[pl.ds(pbase, DCH)], idx_b[c % 2])
            in_h[c] = pltpu.async_copy(
                xfb_hbm.at[pl.ds(src, DCH)], rows_b[c % 2], s_in)
            win_h[c] = pltpu.async_copy(
                wpr_hbm.at[pl.ds(pbase, DCH)], wgt_b[c % 2], s_wi)

        start_in(0)
        for c in range(NCH):
            if c + 1 < NCH:
                if c >= 1:
                    sc_h[c - 1].wait()
                    ws_h[c - 1].wait()
                start_in(c + 1)
            in_h[c].wait()
            win_h[c].wait()
            sc_h[c] = pltpu.async_copy(rows_b[c % 2], buf_hbm.at[idx_b[c % 2]],
                                       s_sc)
            ws_h[c] = pltpu.async_copy(wgt_b[c % 2], wgt_hbm.at[idx_b[c % 2]],
                                       s_ws)
        for j in (NCH - 2, NCH - 1):
            sc_h[j].wait()
            ws_h[j].wait()

    return _k(xfb, dst, wpr)


CB = 2
CAP_BLK = CAP // CB


def _ffn_body(cnt_ref, buf_ref, wgt_ref, w1_ref, w3_ref, w2_ref, out_ref):
    e = pl.program_id(0)
    cb = pl.program_id(1)
    cval = cnt_ref[0, e]
    rows = lax.broadcasted_iota(jnp.int32, (CAP_BLK, 1), 0) + cb * CAP_BLK
    mask = rows < cval
    wcol = wgt_ref[...][:, 0:1]
    w = buf_ref[...]
    lo = lax.bitcast_convert_type(lax.shift_left(w, 16), jnp.float32)
    hi = lax.bitcast_convert_type(w & jnp.int32(-65536), jnp.float32)
    blk = jnp.concatenate([lo, hi], axis=1)
    a = jnp.where(mask, blk * wcol, 0.0)
    h1 = jnp.dot(a, w1_ref[0], preferred_element_type=jnp.float32,
                 precision=lax.Precision.DEFAULT)
    h3 = jnp.dot(a, w3_ref[0], preferred_element_type=jnp.float32,
                 precision=lax.Precision.DEFAULT)
    g = (h1 * jax.nn.sigmoid(h1)) * h3
    out_ref[...] = jnp.dot(g, w2_ref[0], preferred_element_type=jnp.float32,
                           precision=lax.Precision.DEFAULT)


def _ffn(cnt, buf, wgt, w1b, w3b, w2b):
    return pl.pallas_call(
        _ffn_body,
        grid=(E, CB),
        in_specs=[
            pl.BlockSpec(memory_space=pltpu.SMEM),
            pl.BlockSpec((CAP_BLK, H // 2), lambda e, cb: (e * CB + cb, 0)),
            pl.BlockSpec((CAP_BLK, WGT_W), lambda e, cb: (e * CB + cb, 0)),
            pl.BlockSpec((1, H, FFN), lambda e, cb: (e, 0, 0)),
            pl.BlockSpec((1, H, FFN), lambda e, cb: (e, 0, 0)),
            pl.BlockSpec((1, FFN, H), lambda e, cb: (e, 0, 0)),
        ],
        out_specs=pl.BlockSpec((CAP_BLK, H), lambda e, cb: (e * CB + cb, 0)),
        out_shape=jax.ShapeDtypeStruct((R, H), jnp.float32),
    )(cnt, buf, wgt, w1b, w3b, w2b)


TPW = N // NW
CCH = 16


def _combine(eout, gat):
    mesh = plsc.VectorSubcoreMesh(core_axis_name="c", subcore_axis_name="s")
    NCH = TPW // CCH

    @functools.partial(
        pl.kernel,
        out_type=jax.ShapeDtypeStruct((N, H), jnp.float32),
        mesh=mesh,
        scratch_types=[
            pltpu.VMEM((CCH,), jnp.int32),
            pltpu.VMEM((CCH,), jnp.int32),
            pltpu.VMEM((CCH,), jnp.int32),
            pltpu.VMEM((CCH,), jnp.int32),
            pltpu.VMEM((CCH, H), jnp.float32),
            pltpu.VMEM((CCH, H), jnp.float32),
            pltpu.VMEM((CCH, H), jnp.float32),
            pltpu.VMEM((CCH, H), jnp.float32),
            pltpu.SemaphoreType.DMA,
            pltpu.SemaphoreType.DMA,
        ],
    )
    def _k(eout_hbm, gat_hbm, out_hbm, ia0, ia1, ib0, ib1,
           r0a, r0b, r1a, r1b, s_g, s_o):
        wid = lax.axis_index("s") * NC + lax.axis_index("c")
        i0_b = [ia0, ia1]
        i1_b = [ib0, ib1]
        r0_b = [r0a, r0b]
        r1_b = [r1a, r1b]
        g0 = [None] * NCH
        g1 = [None] * NCH
        out_h = [None] * NCH

        def start_g(c):
            tb = wid * TPW + c * CCH
            pltpu.sync_copy(gat_hbm.at[pl.ds(tb, CCH)], i0_b[c % 2])
            pltpu.sync_copy(gat_hbm.at[pl.ds(N + tb, CCH)], i1_b[c % 2])
            g0[c] = pltpu.async_copy(eout_hbm.at[i0_b[c % 2]], r0_b[c % 2], s_g)
            g1[c] = pltpu.async_copy(eout_hbm.at[i1_b[c % 2]], r1_b[c % 2], s_g)

        start_g(0)
        for c in range(NCH):
            if c + 1 < NCH:
                if c >= 1:
                    out_h[c - 1].wait()
                start_g(c + 1)
            g0[c].wait()
            g1[c].wait()
            r0 = r0_b[c % 2]
            r1 = r1_b[c % 2]

            def _row(r, _):
                for h in range(H // 16):
                    sl = pl.ds(h * 16, 16)
                    r0[r, sl] = r0[r, sl] + r1[r, sl]
                return 0

            lax.fori_loop(0, CCH, _row, 0)
            tb = wid * TPW + c * CCH
            out_h[c] = pltpu.async_copy(r0, out_hbm.at[pl.ds(tb, CCH)], s_o)
        for j in (NCH - 2, NCH - 1):
            out_h[j].wait()

    return _k(eout, gat)


def kernel(x, rms_g, Wr, w1, w3, w2):
    xf = x.reshape(N, H)
    g2d = rms_g.reshape(1, H)
    dst, gat, wpr, cnt, aux, xfb = _routing(xf, g2d, Wr)
    buf, wgt = _dispatch(xfb, dst, wpr)
    eout = _ffn(cnt, buf, wgt, w1, w3, w2)
    out = _combine(eout, gat)
    return out.reshape(B, S, H), aux.reshape(())
